# baseline (speedup 1.0000x reference)
import functools

import jax
import jax.numpy as jnp
from jax import lax
from jax.experimental import pallas as pl
from jax.experimental.pallas import tpu as pltpu
from jax.experimental.pallas import tpu_sc as plsc

NFEAT = 56
DIM = 128
CBITS = 6
NCHUNK = 10
TROWS = NCHUNK * 64
OFFPAD = 16
NLANE = 16
NVREG = DIM // NLANE


def _table_body(tab_ref, t_ref):
    r = lax.broadcasted_iota(jnp.int32, (TROWS, NFEAT), 0)
    i = lax.broadcasted_iota(jnp.int32, (TROWS, NFEAT), 1)
    c = r // 64
    b = r % 64
    j = i - c * CBITS
    inch = (j >= 0) & (j < CBITS)
    bit = jnp.right_shift(b, jnp.clip(j, 0, CBITS - 1)) & 1
    tab = tab_ref[...]
    m1 = (inch & (bit == 1)).astype(jnp.float32)
    m0 = (inch & (bit == 0)).astype(jnp.float32)
    dn = (((1,), (0,)), ((), ()))
    t_ref[...] = (
        lax.dot_general(m1, tab[:, 1, :], dn, preferred_element_type=jnp.float32)
        + lax.dot_general(m0, tab[:, 0, :], dn, preferred_element_type=jnp.float32))


def _pack_body(x_ref, w_ref):
    i = lax.broadcasted_iota(jnp.int32, (NFEAT, OFFPAD), 0)
    c = lax.broadcasted_iota(jnp.int32, (NFEAT, OFFPAD), 1)
    p = jnp.where((i // CBITS) == c,
                  jnp.left_shift(DIM, i % CBITS), 0).astype(jnp.float32)
    xf = x_ref[...].astype(jnp.float32)
    wf = lax.dot_general(xf, p, (((1,), (0,)), ((), ())),
                         preferred_element_type=jnp.float32)
    cc = lax.broadcasted_iota(jnp.int32, (x_ref.shape[0], OFFPAD), 1)
    w_ref[...] = wf.astype(jnp.int32) + jnp.where(
        cc < NCHUNK, cc * (64 * DIM), 0)


NCORE = 2
NSUBCORE = 16


@functools.lru_cache(maxsize=None)
def _make_sc(n):
    nw = NCORE * NSUBCORE
    rpw = n // nw
    assert n % nw == 0
    bn = 125
    assert rpw % bn == 0
    nblk = rpw // bn

    def body(woff_hbm, t_hbm, out_hbm, t_v, w_v, o_v):
        wid = lax.axis_index("s") * NCORE + lax.axis_index("c")
        row0 = wid * rpw
        pltpu.sync_copy(t_hbm, t_v)

        def block(blk, carry):
            r0 = row0 + blk * bn
            pltpu.sync_copy(woff_hbm.at[pl.ds(r0 * OFFPAD, bn * OFFPAD)], w_v)

            def row(r, carry2):
                wvec = w_v[pl.ds(r * OFFPAD, NLANE)]
                o0 = wvec[0]
                accs = [t_v[pl.ds(o0 + d * NLANE, NLANE)]
                        for d in range(NVREG)]
                for ci in range(1, NCHUNK):
                    oc = wvec[ci]
                    for d in range(NVREG):
                        accs[d] = accs[d] + t_v[pl.ds(oc + d * NLANE, NLANE)]
                for d in range(NVREG):
                    o_v[pl.ds(r * DIM + d * NLANE, NLANE)] = accs[d]
                return carry2

            lax.fori_loop(0, bn, row, 0)
            pltpu.sync_copy(o_v, out_hbm.at[pl.ds(r0 * DIM, bn * DIM)])
            return carry

        lax.fori_loop(0, nblk, block, 0)

    return pl.kernel(
        body,
        out_type=jax.ShapeDtypeStruct((n * DIM,), jnp.float32),
        mesh=plsc.VectorSubcoreMesh(core_axis_name="c", subcore_axis_name="s",
                                    num_cores=NCORE, num_subcores=NSUBCORE),
        scratch_types=[
            pltpu.VMEM((TROWS * DIM,), jnp.float32),
            pltpu.VMEM((bn * OFFPAD,), jnp.int32),
            pltpu.VMEM((bn * DIM,), jnp.float32),
        ],
    )


def kernel(x, tables):
    n = x.shape[0]
    t = pl.pallas_call(
        _table_body,
        out_shape=jax.ShapeDtypeStruct((TROWS, DIM), jnp.float32),
    )(tables)
    blk = 2000
    assert n % blk == 0
    woff = pl.pallas_call(
        _pack_body,
        grid=(n // blk,),
        in_specs=[pl.BlockSpec((blk, NFEAT), lambda i: (i, 0))],
        out_specs=pl.BlockSpec((blk, OFFPAD), lambda i: (i, 0)),
        out_shape=jax.ShapeDtypeStruct((n, OFFPAD), jnp.int32),
    )(x)
    out = _make_sc(n)(woff.reshape(-1), t.reshape(-1))
    return out.reshape(n, DIM)


# speedup vs baseline: 1.2494x; 1.2494x over previous
import functools

import jax
import jax.numpy as jnp
from jax import lax
from jax.experimental import pallas as pl
from jax.experimental.pallas import tpu as pltpu
from jax.experimental.pallas import tpu_sc as plsc

NFEAT = 56
DIM = 128
CBITS = 7
NCHUNK = 8
CROWS = 1 << CBITS
TROWS = NCHUNK * CROWS
NLANE = 16
NCORE = 2
NSUBCORE = 16


def _table_body(tab_ref, t_ref):
    r = lax.broadcasted_iota(jnp.int32, (TROWS, NFEAT), 0)
    i = lax.broadcasted_iota(jnp.int32, (TROWS, NFEAT), 1)
    c = r // CROWS
    b = r % CROWS
    j = i - c * CBITS
    inch = (j >= 0) & (j < CBITS)
    bit = jnp.right_shift(b, jnp.clip(j, 0, CBITS - 1)) & 1
    tab = tab_ref[...]
    m1 = (inch & (bit == 1)).astype(jnp.float32)
    m0 = (inch & (bit == 0)).astype(jnp.float32)
    dn = (((1,), (0,)), ((), ()))
    t_ref[...] = (
        lax.dot_general(m1, tab[:, 1, :], dn, preferred_element_type=jnp.float32)
        + lax.dot_general(m0, tab[:, 0, :], dn, preferred_element_type=jnp.float32))


def _pack_body(x_ref, w_ref):
    i = lax.broadcasted_iota(jnp.int32, (NFEAT, NCHUNK), 0)
    c = lax.broadcasted_iota(jnp.int32, (NFEAT, NCHUNK), 1)
    p = jnp.where((i // CBITS) == c,
                  jnp.left_shift(1, i % CBITS), 0).astype(jnp.float32)
    xf = x_ref[...].astype(jnp.float32)
    wf = lax.dot_general(xf, p, (((1,), (0,)), ((), ())),
                         preferred_element_type=jnp.float32)
    cc = lax.broadcasted_iota(jnp.int32, (x_ref.shape[0], NCHUNK), 1)
    w_ref[...] = wf.astype(jnp.int32) + cc * CROWS


@functools.lru_cache(maxsize=None)
def _make_sc(n):
    nw = NCORE * NSUBCORE
    rpw = n // nw
    assert n % nw == 0
    bn = 125
    assert rpw % bn == 0
    nblk = rpw // bn
    npair = bn // 2

    def body(woff_hbm, t_hbm, out_hbm, t_v, w_v, o_v):
        wid = lax.axis_index("s") * NCORE + lax.axis_index("c")
        row0 = wid * rpw
        pltpu.sync_copy(t_hbm, t_v)

        hi_mask = jnp.int32(-65536)

        def unpack2(v):
            a = lax.bitcast_convert_type(lax.shift_left(v, 16), jnp.float32)
            b = lax.bitcast_convert_type(v & hi_mask, jnp.float32)
            return a, b

        def accum_row(wvec, lane0, r):
            o0 = wvec[lane0] * (DIM // 2)
            accs = []
            for g in range(4):
                a, b = unpack2(t_v[pl.ds(o0 + g * NLANE, NLANE)])
                accs.append([a, b])
            for ci in range(1, NCHUNK):
                oc = wvec[lane0 + ci] * (DIM // 2)
                for g in range(4):
                    a, b = unpack2(t_v[pl.ds(oc + g * NLANE, NLANE)])
                    accs[g][0] += a
                    accs[g][1] += b
            for g in range(4):
                o_v[pl.ds(r * DIM + g * 32, NLANE)] = accs[g][0]
                o_v[pl.ds(r * DIM + g * 32 + NLANE, NLANE)] = accs[g][1]

        def block(blk, carry):
            r0 = row0 + blk * bn
            pltpu.sync_copy(woff_hbm.at[pl.ds(r0 * NCHUNK, bn * NCHUNK)],
                            w_v.at[pl.ds(0, bn * NCHUNK)])

            def pair(p2, carry2):
                wvec = w_v[pl.ds(p2 * NLANE, NLANE)]
                accum_row(wvec, 0, 2 * p2)
                accum_row(wvec, NCHUNK, 2 * p2 + 1)
                return carry2

            lax.fori_loop(0, npair, pair, 0)
            wtail = w_v[pl.ds((bn - 1) * NCHUNK, NLANE)]
            accum_row(wtail, 0, bn - 1)
            pltpu.sync_copy(o_v, out_hbm.at[pl.ds(r0 * DIM, bn * DIM)])
            return carry

        lax.fori_loop(0, nblk, block, 0)

    return pl.kernel(
        body,
        out_type=jax.ShapeDtypeStruct((n * DIM,), jnp.float32),
        mesh=plsc.VectorSubcoreMesh(core_axis_name="c", subcore_axis_name="s",
                                    num_cores=NCORE, num_subcores=NSUBCORE),
        scratch_types=[
            pltpu.VMEM((TROWS * DIM // 2,), jnp.int32),
            pltpu.VMEM((bn * NCHUNK + NCHUNK,), jnp.int32),
            pltpu.VMEM((bn * DIM,), jnp.float32),
        ],
    )


def kernel(x, tables):
    n = x.shape[0]
    t = pl.pallas_call(
        _table_body,
        out_shape=jax.ShapeDtypeStruct((TROWS, DIM), jnp.float32),
    )(tables)
    tb = lax.bitcast_convert_type(
        t.reshape(TROWS, 4, 2, NLANE).astype(jnp.bfloat16),
        jnp.uint16).astype(jnp.uint32)
    t_pk = lax.bitcast_convert_type(
        tb[:, :, 0, :] | (tb[:, :, 1, :] << 16), jnp.int32)
    blk = 2000
    assert n % blk == 0
    woff = pl.pallas_call(
        _pack_body,
        grid=(n // blk,),
        in_specs=[pl.BlockSpec((blk, NFEAT), lambda i: (i, 0))],
        out_specs=pl.BlockSpec((blk, NCHUNK), lambda i: (i, 0)),
        out_shape=jax.ShapeDtypeStruct((n, NCHUNK), jnp.int32),
    )(x)
    out = _make_sc(n)(woff.reshape(-1), t_pk.reshape(-1))
    return out.reshape(n, DIM)
